# baseline (speedup 1.0000x reference)
import functools

import jax
import jax.numpy as jnp
from jax import lax
from jax.experimental import pallas as pl
from jax.experimental.pallas import tpu as pltpu
from jax.experimental.pallas import tpu_sc as plsc

V = 1000000
D = 64
B = 425984
NC = 2
NS = 16
NW = NC * NS
BPW = B // NW
TCOLS = B // 128
TPW = TCOLS // NW
CHUNK = 512
CT = CHUNK // 128
NCHUNK = BPW // CHUNK

_mesh = plsc.VectorSubcoreMesh(core_axis_name="c", subcore_axis_name="s")


@functools.partial(
    pl.kernel,
    mesh=_mesh,
    out_type=jax.ShapeDtypeStruct((8, TCOLS, 8, 128), jnp.float32),
    scratch_types=[
        pltpu.VMEM((CHUNK,), jnp.int32),
        pltpu.VMEM((CHUNK,), jnp.int32),
        pltpu.VMEM((CHUNK, 128), jnp.float32),
        pltpu.VMEM((8, CT, 8, 128), jnp.float32),
        pltpu.SemaphoreType.DMA,
        pltpu.SemaphoreType.DMA,
    ],
    compiler_params=pltpu.CompilerParams(needs_layout_passes=False),
)
def _gather(x2_hbm, y_hbm, out_hbm, idx_v, q_v, rows_v, tiles_v,
            sem_g, sem_o):
    wid = lax.axis_index("s") * NC + lax.axis_index("c")
    base = wid * BPW
    tbase = wid * TPW

    lanes = lax.iota(jnp.int32, 16)
    tr_vecs = [(lanes + 16 * v) >> 3 for v in range(4)]
    k_vecs = [(lanes + 16 * v) & 7 for v in range(4)]

    for ci in range(NCHUNK):
        off = base + ci * CHUNK
        pltpu.sync_copy(y_hbm.at[pl.ds(off, CHUNK)], idx_v)
        for b in range(CHUNK // 16):
            q_v[pl.ds(16 * b, 16)] = idx_v[pl.ds(16 * b, 16)] >> 1
        pltpu.async_copy(x2_hbm.at[q_v], rows_v, sem_g).wait()

        def body(jo, _):
            jbase = 16 * jo
            halves = (idx_v[pl.ds(jbase, 16)] & 1) * 64
            for i in range(16):
                j = jbase + i
                half = halves[i]
                c4_vec = jnp.zeros((16,), jnp.int32) + (j >> 7)
                m_vec = jnp.zeros((16,), jnp.int32) + (j & 127)
                for v in range(4):
                    val = rows_v[j, pl.ds(half + 16 * v, 16)]
                    plsc.store_scatter(
                        tiles_v, [tr_vecs[v], c4_vec, k_vecs[v], m_vec], val)
            return 0

        lax.fori_loop(0, CHUNK // 16, body, 0)
        pltpu.async_copy(
            tiles_v, out_hbm.at[:, pl.ds(tbase + ci * CT, CT)], sem_o
        ).wait()


def kernel(x, y):
    x2 = x.reshape(V // 2, 2 * D)
    out4 = _gather(x2, y)
    return out4.transpose(1, 3, 0, 2).reshape(B, D)
